# baseline (speedup 1.0000x reference)
import functools

import jax
import jax.numpy as jnp
from jax import lax
from jax.experimental import pallas as pl
from jax.experimental.pallas import tpu as pltpu
from jax.experimental.pallas import tpu_sc as plsc

NC = 2
NS = 16
NW = NC * NS

SAMP = 8
LANES = 16
XPAD = 128


@functools.partial(jax.jit, static_argnames=("B", "F", "V", "H", "f0", "Fk"))
def _embed_gather(Xp, tables, *, B, F, V, H, f0, Fk):
    spw = B // NW
    n_blocks = spw // SAMP
    rows_pb = SAMP * Fk
    assert spw % SAMP == 0 and n_blocks % 2 == 0
    mesh = plsc.VectorSubcoreMesh(
        core_axis_name="c", subcore_axis_name="s",
        num_cores=NC, num_subcores=NS)

    @functools.partial(
        pl.kernel,
        out_type=jax.ShapeDtypeStruct((B, Fk * H), jnp.float32),
        mesh=mesh,
        scratch_types=[
            pltpu.VMEM((2, SAMP, XPAD), jnp.int32),
            pltpu.VMEM((2, rows_pb, H), jnp.float32),
            pltpu.VMEM((2, SAMP, Fk * H), jnp.float32),
            [pltpu.SemaphoreType.DMA] * 2,
            [pltpu.SemaphoreType.DMA] * 2,
            [pltpu.SemaphoreType.DMA] * 2,
        ],
    )
    def k(x_hbm, tab_hbm, out_hbm, xbuf, rows, obuf, xsems, gsems, osems):
        wid = lax.axis_index("s") * NC + lax.axis_index("c")
        sbase = wid * spw

        for s in range(2):
            pltpu.async_copy(
                x_hbm.at[pl.ds(sbase + s * SAMP, SAMP)], xbuf.at[s],
                xsems[s])

        def block(n, s):
            b0 = sbase + n * SAMP

            @pl.when(n >= 2)
            def _drain_prev_write():
                pltpu.make_async_copy(
                    obuf.at[s], out_hbm.at[pl.ds(b0 - 2 * SAMP, SAMP)],
                    osems[s]).wait()

            pltpu.make_async_copy(
                x_hbm.at[pl.ds(b0, SAMP)], xbuf.at[s], xsems[s]).wait()

            def fire_rows(i, carry):
                j0 = i * Fk
                va = xbuf[s, i, pl.ds(0, LANES)]
                vb = xbuf[s, i, pl.ds(LANES, LANES)]
                for fl in range(Fk):
                    f = f0 + fl
                    row = va[f] if f < LANES else vb[f - LANES]
                    pltpu.async_copy(
                        tab_hbm.at[fl, pl.ds(row, 1), :],
                        rows.at[s, pl.ds(j0 + fl, 1), :],
                        gsems[s])
                return carry

            lax.fori_loop(0, SAMP, fire_rows, 0)

            @pl.when(n + 2 < n_blocks)
            def _prefetch_x():
                pltpu.async_copy(
                    x_hbm.at[pl.ds(b0 + 2 * SAMP, SAMP)], xbuf.at[s],
                    xsems[s])

            pltpu.make_async_copy(
                tab_hbm.at[0, pl.ds(0, rows_pb), :], rows.at[s],
                gsems[s]).wait()

            def assemble(i, carry):
                j0 = i * Fk
                for fl in range(Fk):
                    for t in range(H // LANES):
                        obuf[s, i, pl.ds(fl * H + t * LANES, LANES)] = (
                            rows[s, j0 + fl, pl.ds(t * LANES, LANES)])
                return carry

            lax.fori_loop(0, SAMP, assemble, 0)
            pltpu.async_copy(
                obuf.at[s], out_hbm.at[pl.ds(b0, SAMP)], osems[s])

        def pair(p, carry):
            block(2 * p, 0)
            block(2 * p + 1, 1)
            return carry

        lax.fori_loop(0, n_blocks // 2, pair, 0)
        for s in range(2):
            pltpu.make_async_copy(
                obuf.at[s],
                out_hbm.at[pl.ds(sbase + (n_blocks - 2 + s) * SAMP, SAMP)],
                osems[s]).wait()

    return k(Xp, tables)


def kernel(X, tables):
    F, V, H = tables.shape
    B = X.shape[0]
    Fk = F // 2
    Xp = jnp.pad(X.astype(jnp.int32), ((0, 0), (0, XPAD - F)))
    halves = [
        _embed_gather(Xp, tables[f0:f0 + Fk], B=B, F=F, V=V, H=H,
                      f0=f0, Fk=Fk)
        for f0 in (0, Fk)
    ]
    return jnp.concatenate(halves, axis=1)


# speedup vs baseline: 1.3208x; 1.3208x over previous
import functools

import jax
import jax.numpy as jnp
from jax import lax
from jax.experimental import pallas as pl
from jax.experimental.pallas import tpu as pltpu
from jax.experimental.pallas import tpu_sc as plsc

NC = 2
NS = 16
NW = NC * NS

SAMP = 8
LANES = 16
XPAD = 128


@functools.partial(jax.jit, static_argnames=("B", "F", "V", "H"))
def _embed_gather(Xp, tables, *, B, F, V, H):
    spw = B // NW
    n_blocks = spw // SAMP
    rows_pb = SAMP * F
    assert spw % SAMP == 0 and n_blocks % 2 == 0
    mesh = plsc.VectorSubcoreMesh(
        core_axis_name="c", subcore_axis_name="s",
        num_cores=NC, num_subcores=NS)

    @functools.partial(
        pl.kernel,
        out_type=jax.ShapeDtypeStruct((B, F * H), jnp.float32),
        mesh=mesh,
        scratch_types=[
            pltpu.VMEM((2, SAMP, XPAD), jnp.int32),
            pltpu.VMEM((2, rows_pb, H), jnp.float32),
            pltpu.VMEM((2, SAMP, F * H), jnp.float32),
            [pltpu.SemaphoreType.DMA] * 2,
            [pltpu.SemaphoreType.DMA] * 2,
            [pltpu.SemaphoreType.DMA] * 2,
        ],
    )
    def k(x_hbm, tab_hbm, out_hbm, xbuf, rows, obuf, xsems, gsems, osems):
        wid = lax.axis_index("s") * NC + lax.axis_index("c")
        sbase = wid * spw

        for s in range(2):
            pltpu.async_copy(
                x_hbm.at[pl.ds(sbase + s * SAMP, SAMP)], xbuf.at[s],
                xsems[s])

        def block(n, s):
            b0 = sbase + n * SAMP

            @pl.when(n >= 2)
            def _drain_prev_write():
                pltpu.make_async_copy(
                    obuf.at[s], out_hbm.at[pl.ds(b0 - 2 * SAMP, SAMP)],
                    osems[s]).wait()

            pltpu.make_async_copy(
                x_hbm.at[pl.ds(b0, SAMP)], xbuf.at[s], xsems[s]).wait()

            def fire_rows(i, carry):
                j0 = i * F
                va = xbuf[s, i, pl.ds(0, LANES)]
                vb = xbuf[s, i, pl.ds(LANES, LANES)]
                for f in range(F):
                    row = va[f] if f < LANES else vb[f - LANES]
                    pltpu.async_copy(
                        tab_hbm.at[f, pl.ds(row, 1), :],
                        rows.at[s, pl.ds(j0 + f, 1), :],
                        gsems[s])
                return carry

            lax.fori_loop(0, SAMP, fire_rows, 0)

            @pl.when(n + 2 < n_blocks)
            def _prefetch_x():
                pltpu.async_copy(
                    x_hbm.at[pl.ds(b0 + 2 * SAMP, SAMP)], xbuf.at[s],
                    xsems[s])

            pltpu.make_async_copy(
                tab_hbm.at[0, pl.ds(0, rows_pb), :], rows.at[s],
                gsems[s]).wait()

            def assemble(i, carry):
                j0 = i * F
                for f in range(F):
                    for t in range(H // LANES):
                        obuf[s, i, pl.ds(f * H + t * LANES, LANES)] = (
                            rows[s, j0 + f, pl.ds(t * LANES, LANES)])
                return carry

            lax.fori_loop(0, SAMP, assemble, 0)
            pltpu.async_copy(
                obuf.at[s], out_hbm.at[pl.ds(b0, SAMP)], osems[s])

        def pair(p, carry):
            block(2 * p, 0)
            block(2 * p + 1, 1)
            return carry

        lax.fori_loop(0, n_blocks // 2, pair, 0)
        for s in range(2):
            pltpu.make_async_copy(
                obuf.at[s],
                out_hbm.at[pl.ds(sbase + (n_blocks - 2 + s) * SAMP, SAMP)],
                osems[s]).wait()

    return k(Xp, tables)


def kernel(X, tables):
    F, V, H = tables.shape
    B = X.shape[0]
    Xp = jnp.pad(X.astype(jnp.int32), ((0, 0), (0, XPAD - F)))
    return _embed_gather(Xp, tables, B=B, F=F, V=V, H=H)
